# baseline (speedup 1.0000x reference)
import dataclasses
import functools

import jax
import jax.numpy as jnp
from jax import lax
from jax.experimental import pallas as pl
from jax.experimental.pallas import tpu as pltpu
from jax.experimental.pallas import tpu_sc as plsc

_TOPK = 5
_NC = 2
_NS = 16
_L = 16
_NEG = -3.0e38


def _tc_reduce_body(a_ref, q_ref, s_ref, qs_ref):
    c = pl.program_id(1)

    @pl.when(c == 0)
    def _():
        s_ref[...] = jnp.zeros_like(s_ref)
        qs_ref[...] = jnp.zeros_like(qs_ref)

    s_ref[...] += jnp.sum(a_ref[...], axis=1, keepdims=True)
    qs_ref[...] += jnp.sum(q_ref[...], axis=1, keepdims=True)


def _tc_reduce(attn3, question_emb, n_chunks, b_lo, nb):
    _, R, Lc = attn3.shape
    _, Lq, D = question_emb.shape
    rc = R // n_chunks
    qc = Lq // n_chunks
    return pl.pallas_call(
        _tc_reduce_body,
        grid=(nb, n_chunks),
        in_specs=[
            pl.BlockSpec((1, rc, Lc), lambda b, c: (b + b_lo, c, 0)),
            pl.BlockSpec((1, qc, D), lambda b, c: (b + b_lo, c, 0)),
        ],
        out_specs=[
            pl.BlockSpec((1, 1, Lc), lambda b, c: (b, 0, 0)),
            pl.BlockSpec((1, 1, D), lambda b, c: (b, 0, 0)),
        ],
        out_shape=[
            jax.ShapeDtypeStruct((nb, 1, Lc), jnp.float32),
            jax.ShapeDtypeStruct((nb, 1, D), jnp.float32),
        ],
    )(attn3, question_emb)


def _lanes(scalars, fill, iv, dtype):
    v = jnp.full((_L,), fill, dtype)
    for j, x in enumerate(scalars):
        v = jnp.where(iv == j, x, v)
    return v


def _top5_scan(load_chunk, n_chunks, iv, unroll):
    vals, idxs = [], []
    for p in range(_TOPK):
        prev = (vals[-1], idxs[-1]) if p else None

        def chunk(ci, carry, prev=prev):
            bv, bi = carry
            v, gi = load_chunk(ci)
            if prev is not None:
                pv, pi = prev
                keep = (v < pv) | ((v == pv) & (gi > pi))
                v = jnp.where(keep, v, jnp.float32(_NEG))
            m = v > bv
            return jnp.where(m, v, bv), jnp.where(m, gi, bi)

        carry = (jnp.full((_L,), _NEG, jnp.float32),
                 jnp.zeros((_L,), jnp.int32))
        bv, bi = lax.fori_loop(0, n_chunks, chunk, carry, unroll=unroll)
        mx = jnp.max(bv)
        idxs.append(jnp.min(jnp.where(bv == mx, bi, jnp.int32(1 << 30))))
        vals.append(mx)
    return vals, idxs


def _sc_body(B, Lc, D, b_lo, nctx, s_hbm, q_hbm, ctx_hbm, out_hbm,
             s_v, q_v, idx_v, rows_v, o_v, sem):
    wid = lax.axis_index("s") * _NC + lax.axis_index("c")

    @pl.when(wid < B)
    def _():
        b = wid
        iv = lax.iota(jnp.int32, _L)

        pltpu.sync_copy(s_hbm.at[pl.ds(b * Lc, Lc)], s_v)
        qcp = pltpu.async_copy(q_hbm.at[pl.ds(b * D, D)], q_v, sem)

        def load_local(ci):
            return s_v[pl.ds(ci * _L, _L)], ci * _L + iv

        _v, midx = _top5_scan(load_local, Lc // _L, iv, unroll=8)

        gidx = _lanes(midx, 0, iv, jnp.int32)
        idx_v[...] = jnp.clip(gidx + (b + b_lo) * Lc, 0, nctx - 1)
        pltpu.sync_copy(ctx_hbm.at[idx_v], rows_v)
        qcp.wait()

        zero = jnp.zeros((_L,), jnp.float32)

        def dchunk(ci, carry):
            qq = carry[0]
            dots = list(carry[1])
            nrm = list(carry[2])
            qv = q_v[pl.ds(ci * _L, _L)]
            qq = qq + qv * qv
            for j in range(_TOPK):
                rv = rows_v[j, pl.ds(ci * _L, _L)]
                dots[j] = dots[j] + qv * rv
                nrm[j] = nrm[j] + rv * rv
            return qq, tuple(dots), tuple(nrm)

        qq, dots, nrm = lax.fori_loop(
            0, D // _L, dchunk,
            (zero, (zero,) * _TOPK, (zero,) * _TOPK),
            unroll=4)

        qqs = jnp.sum(qq)
        dotv = _lanes([jnp.sum(d) for d in dots], 0.0, iv, jnp.float32)
        ccv = _lanes([jnp.sum(n) for n in nrm], 1.0, iv, jnp.float32)

        s2 = ccv * qqs
        y = lax.bitcast_convert_type(
            jnp.int32(0x5F3759DF) - (lax.bitcast_convert_type(s2, jnp.int32) >> 1),
            jnp.float32)
        for _ in range(4):
            y = y * (jnp.float32(1.5) - jnp.float32(0.5) * s2 * y * y)
        denom = jnp.maximum(s2 * y, jnp.float32(1e-8))
        sim = dotv / denom
        o_v[...] = jnp.where(iv < _TOPK, sim, jnp.float32(0.0))
        pltpu.sync_copy(o_v, out_hbm.at[b])


def _sc_stage(sums, qsums, ctx2d, b_lo):
    B, Lc = sums.shape
    D = qsums.shape[1]
    sums = sums.reshape(B * Lc)
    qsums = qsums.reshape(B * D)
    mesh = plsc.VectorSubcoreMesh(core_axis_name="c", subcore_axis_name="s")
    body = functools.partial(_sc_body, B, Lc, D, b_lo, ctx2d.shape[0])
    cp = pltpu.CompilerParams()
    if "needs_layout_passes" in pltpu.CompilerParams.__dataclass_fields__:
        cp = dataclasses.replace(cp, needs_layout_passes=False)
    kfn = pl.kernel(
        body,
        out_type=jax.ShapeDtypeStruct((B, _L), jnp.float32),
        mesh=mesh,
        compiler_params=cp,
        scratch_types=[
            pltpu.VMEM((Lc,), jnp.float32),
            pltpu.VMEM((D,), jnp.float32),
            pltpu.VMEM((_L,), jnp.int32),
            pltpu.VMEM((_L, D), jnp.float32),
            pltpu.VMEM((_L,), jnp.float32),
            pltpu.SemaphoreType.DMA,
        ],
    )
    return kfn(sums, qsums, ctx2d)


def kernel(question_emb, context_emb, cross_attn_weights):
    B, Lq, D = question_emb.shape
    Lc = context_emb.shape[1]
    attn3 = cross_attn_weights.reshape(B, -1, Lc)
    ctx2d = context_emb.reshape(B * Lc, D)
    nb = B // 2
    sims = []
    for b_lo in (0, nb):
        s_h, q_h = _tc_reduce(attn3, question_emb, n_chunks=4,
                              b_lo=b_lo, nb=nb)
        sims.append(_sc_stage(s_h.reshape(nb, Lc), q_h.reshape(nb, D),
                              ctx2d, b_lo))
    sims = jnp.concatenate(sims, axis=0)
    per_batch = 1.0 - jnp.sum(sims, axis=1) / _TOPK
    return jnp.mean(per_batch)


# speedup vs baseline: 1.0217x; 1.0217x over previous
import dataclasses
import functools

import jax
import jax.numpy as jnp
from jax import lax
from jax.experimental import pallas as pl
from jax.experimental.pallas import tpu as pltpu
from jax.experimental.pallas import tpu_sc as plsc

_TOPK = 5
_NC = 2
_NS = 16
_L = 16
_NEG = -3.0e38


def _tc_reduce_body(a_ref, q_ref, s_ref, qs_ref):
    c = pl.program_id(1)

    @pl.when(c == 0)
    def _():
        s_ref[...] = jnp.zeros_like(s_ref)
        qs_ref[...] = jnp.zeros_like(qs_ref)

    s_ref[...] += jnp.sum(a_ref[...], axis=1, keepdims=True)
    qs_ref[...] += jnp.sum(q_ref[...], axis=1, keepdims=True)


def _tc_reduce(attn3, question_emb, n_chunks, b_lo, nb):
    _, R, Lc = attn3.shape
    _, Lq, D = question_emb.shape
    rc = R // n_chunks
    qc = Lq // n_chunks
    return pl.pallas_call(
        _tc_reduce_body,
        grid=(nb, n_chunks),
        in_specs=[
            pl.BlockSpec((1, rc, Lc), lambda b, c: (b + b_lo, c, 0)),
            pl.BlockSpec((1, qc, D), lambda b, c: (b + b_lo, c, 0)),
        ],
        out_specs=[
            pl.BlockSpec((1, 1, Lc), lambda b, c: (b, 0, 0)),
            pl.BlockSpec((1, 1, D), lambda b, c: (b, 0, 0)),
        ],
        out_shape=[
            jax.ShapeDtypeStruct((nb, 1, Lc), jnp.float32),
            jax.ShapeDtypeStruct((nb, 1, D), jnp.float32),
        ],
    )(attn3, question_emb)


def _lanes(scalars, fill, iv, dtype):
    v = jnp.full((_L,), fill, dtype)
    for j, x in enumerate(scalars):
        v = jnp.where(iv == j, x, v)
    return v


def _top5_scan(load_chunk, n_chunks, iv, unroll):
    vals, idxs = [], []
    for p in range(_TOPK):
        prev = (vals[-1], idxs[-1]) if p else None

        def chunk(ci, carry, prev=prev):
            bv, bi = carry
            v, gi = load_chunk(ci)
            if prev is not None:
                pv, pi = prev
                keep = (v < pv) | ((v == pv) & (gi > pi))
                v = jnp.where(keep, v, jnp.float32(_NEG))
            m = v > bv
            return jnp.where(m, v, bv), jnp.where(m, gi, bi)

        carry = (jnp.full((_L,), _NEG, jnp.float32),
                 jnp.zeros((_L,), jnp.int32))
        bv, bi = lax.fori_loop(0, n_chunks, chunk, carry, unroll=unroll)
        mx = jnp.max(bv)
        idxs.append(jnp.min(jnp.where(bv == mx, bi, jnp.int32(1 << 30))))
        vals.append(mx)
    return vals, idxs


def _sc_body(B, Lc, D, b_lo, nctx, s_hbm, q_hbm, ctx_hbm, out_hbm,
             s_v, q_v, idx_v, rows_v, o_v, sem):
    wid = lax.axis_index("s") * _NC + lax.axis_index("c")

    @pl.when(wid < B)
    def _():
        b = wid
        iv = lax.iota(jnp.int32, _L)

        pltpu.sync_copy(s_hbm.at[pl.ds(b * Lc, Lc)], s_v)
        qcp = pltpu.async_copy(q_hbm.at[pl.ds(b * D, D)], q_v, sem)

        def load_local(ci):
            return s_v[pl.ds(ci * _L, _L)], ci * _L + iv

        _v, midx = _top5_scan(load_local, Lc // _L, iv, unroll=16)

        gidx = _lanes(midx, 0, iv, jnp.int32)
        idx_v[...] = jnp.clip(gidx + (b + b_lo) * Lc, 0, nctx - 1)
        pltpu.sync_copy(ctx_hbm.at[idx_v], rows_v)
        qcp.wait()

        zero = jnp.zeros((_L,), jnp.float32)

        def dchunk(ci, carry):
            qq = carry[0]
            dots = list(carry[1])
            nrm = list(carry[2])
            qv = q_v[pl.ds(ci * _L, _L)]
            qq = qq + qv * qv
            for j in range(_TOPK):
                rv = rows_v[j, pl.ds(ci * _L, _L)]
                dots[j] = dots[j] + qv * rv
                nrm[j] = nrm[j] + rv * rv
            return qq, tuple(dots), tuple(nrm)

        qq, dots, nrm = lax.fori_loop(
            0, D // _L, dchunk,
            (zero, (zero,) * _TOPK, (zero,) * _TOPK),
            unroll=4)

        qqs = jnp.sum(qq)
        dotv = _lanes([jnp.sum(d) for d in dots], 0.0, iv, jnp.float32)
        ccv = _lanes([jnp.sum(n) for n in nrm], 1.0, iv, jnp.float32)

        s2 = ccv * qqs
        y = lax.bitcast_convert_type(
            jnp.int32(0x5F3759DF) - (lax.bitcast_convert_type(s2, jnp.int32) >> 1),
            jnp.float32)
        for _ in range(4):
            y = y * (jnp.float32(1.5) - jnp.float32(0.5) * s2 * y * y)
        denom = jnp.maximum(s2 * y, jnp.float32(1e-8))
        sim = dotv / denom
        o_v[...] = jnp.where(iv < _TOPK, sim, jnp.float32(0.0))
        pltpu.sync_copy(o_v, out_hbm.at[b])


def _sc_stage(sums, qsums, ctx2d, b_lo):
    B, Lc = sums.shape
    D = qsums.shape[1]
    sums = sums.reshape(B * Lc)
    qsums = qsums.reshape(B * D)
    mesh = plsc.VectorSubcoreMesh(core_axis_name="c", subcore_axis_name="s")
    body = functools.partial(_sc_body, B, Lc, D, b_lo, ctx2d.shape[0])
    cp = pltpu.CompilerParams()
    if "needs_layout_passes" in pltpu.CompilerParams.__dataclass_fields__:
        cp = dataclasses.replace(cp, needs_layout_passes=False)
    kfn = pl.kernel(
        body,
        out_type=jax.ShapeDtypeStruct((B, _L), jnp.float32),
        mesh=mesh,
        compiler_params=cp,
        scratch_types=[
            pltpu.VMEM((Lc,), jnp.float32),
            pltpu.VMEM((D,), jnp.float32),
            pltpu.VMEM((_L,), jnp.int32),
            pltpu.VMEM((_L, D), jnp.float32),
            pltpu.VMEM((_L,), jnp.float32),
            pltpu.SemaphoreType.DMA,
        ],
    )
    return kfn(sums, qsums, ctx2d)


def kernel(question_emb, context_emb, cross_attn_weights):
    B, Lq, D = question_emb.shape
    Lc = context_emb.shape[1]
    attn3 = cross_attn_weights.reshape(B, -1, Lc)
    ctx2d = context_emb.reshape(B * Lc, D)
    s_h, q_h = _tc_reduce(attn3, question_emb, n_chunks=4, b_lo=0, nb=B)
    sims = _sc_stage(s_h.reshape(B, Lc), q_h.reshape(B, D),
                     ctx2d, 0)
    per_batch = 1.0 - jnp.sum(sims, axis=1) / _TOPK
    return jnp.mean(per_batch)


# speedup vs baseline: 1.0436x; 1.0214x over previous
import dataclasses
import functools

import jax
import jax.numpy as jnp
from jax import lax
from jax.experimental import pallas as pl
from jax.experimental.pallas import tpu as pltpu
from jax.experimental.pallas import tpu_sc as plsc

_TOPK = 5
_NC = 2
_NS = 16
_L = 16
_NEG = -3.0e38


def _tc_reduce_body(a_ref, q_ref, s_ref, qs_ref):
    c = pl.program_id(1)

    @pl.when(c == 0)
    def _():
        s_ref[...] = jnp.zeros_like(s_ref)
        qs_ref[...] = jnp.zeros_like(qs_ref)

    s_ref[...] += jnp.sum(a_ref[...], axis=1, keepdims=True)
    qs_ref[...] += jnp.sum(q_ref[...], axis=1, keepdims=True)


def _tc_reduce(attn3, question_emb, n_chunks, b_lo, nb):
    _, R, Lc = attn3.shape
    _, Lq, D = question_emb.shape
    rc = R // n_chunks
    qc = Lq // n_chunks
    return pl.pallas_call(
        _tc_reduce_body,
        grid=(nb, n_chunks),
        in_specs=[
            pl.BlockSpec((1, rc, Lc), lambda b, c: (b + b_lo, c, 0)),
            pl.BlockSpec((1, qc, D), lambda b, c: (b + b_lo, c, 0)),
        ],
        out_specs=[
            pl.BlockSpec((1, 1, Lc), lambda b, c: (b, 0, 0)),
            pl.BlockSpec((1, 1, D), lambda b, c: (b, 0, 0)),
        ],
        out_shape=[
            jax.ShapeDtypeStruct((nb, 1, Lc), jnp.float32),
            jax.ShapeDtypeStruct((nb, 1, D), jnp.float32),
        ],
    )(attn3, question_emb)


def _lanes(scalars, fill, iv, dtype):
    v = jnp.full((_L,), fill, dtype)
    for j, x in enumerate(scalars):
        v = jnp.where(iv == j, x, v)
    return v


def _grouped_top5(s_v, Lc, iv):
    group, cpg = 64, 4
    ng = Lc // group
    nr = ng // _L
    neg = jnp.float32(_NEG)
    big = jnp.int32(1 << 30)

    gm = []
    for r in range(nr):
        greg = jnp.full((_L,), neg, jnp.float32)
        for j in range(_L):
            g = r * _L + j
            m = s_v[pl.ds(g * group, _L)]
            for k in range(1, cpg):
                m = jnp.maximum(m, s_v[pl.ds(g * group + k * _L, _L)])
            greg = jnp.where(iv == j, jnp.max(m), greg)
        gm.append(greg)

    vals, idxs = [], []
    for p in range(_TOPK):
        prev = (vals[-1], idxs[-1]) if p else None
        mall = gm[0]
        for r in range(1, nr):
            mall = jnp.maximum(mall, gm[r])
        mx = jnp.max(mall)
        gsel = big
        for r in range(nr):
            gsel = jnp.minimum(
                gsel, jnp.min(jnp.where(gm[r] == mx, r * _L + iv, big)))
        base = gsel * group
        bv = jnp.full((_L,), neg, jnp.float32)
        bi = jnp.zeros((_L,), jnp.int32)
        for k in range(cpg):
            v = s_v[pl.ds(base + k * _L, _L)]
            gi = base + k * _L + iv
            if prev is not None:
                pv, pi = prev
                keep = (v < pv) | ((v == pv) & (gi > pi))
                v = jnp.where(keep, v, neg)
            m = v > bv
            bv = jnp.where(m, v, bv)
            bi = jnp.where(m, gi, bi)
        mv = jnp.max(bv)
        mi = jnp.min(jnp.where(bv == mv, bi, big))
        vals.append(mv)
        idxs.append(mi)
        nv = jnp.full((_L,), neg, jnp.float32)
        for k in range(cpg):
            v = s_v[pl.ds(base + k * _L, _L)]
            gi = base + k * _L + iv
            keep = (v < mv) | ((v == mv) & (gi > mi))
            nv = jnp.maximum(nv, jnp.where(keep, v, neg))
        gnew = jnp.max(nv)
        gr = gsel // _L
        gl = gsel % _L
        for r in range(nr):
            gm[r] = jnp.where((gr == r) & (iv == gl), gnew, gm[r])
    return vals, idxs


def _sc_body(B, Lc, D, b_lo, nctx, s_hbm, q_hbm, ctx_hbm, out_hbm,
             s_v, q_v, idx_v, rows_v, o_v, sem):
    wid = lax.axis_index("s") * _NC + lax.axis_index("c")

    @pl.when(wid < B)
    def _():
        b = wid
        iv = lax.iota(jnp.int32, _L)

        pltpu.sync_copy(s_hbm.at[pl.ds(b * Lc, Lc)], s_v)
        qcp = pltpu.async_copy(q_hbm.at[pl.ds(b * D, D)], q_v, sem)

        _v, midx = _grouped_top5(s_v, Lc, iv)

        gidx = _lanes(midx, 0, iv, jnp.int32)
        idx_v[...] = jnp.clip(gidx + (b + b_lo) * Lc, 0, nctx - 1)
        pltpu.sync_copy(ctx_hbm.at[idx_v], rows_v)
        qcp.wait()

        zero = jnp.zeros((_L,), jnp.float32)

        def dchunk(ci, carry):
            qq = carry[0]
            dots = list(carry[1])
            nrm = list(carry[2])
            qv = q_v[pl.ds(ci * _L, _L)]
            qq = qq + qv * qv
            for j in range(_TOPK):
                rv = rows_v[j, pl.ds(ci * _L, _L)]
                dots[j] = dots[j] + qv * rv
                nrm[j] = nrm[j] + rv * rv
            return qq, tuple(dots), tuple(nrm)

        qq, dots, nrm = lax.fori_loop(
            0, D // _L, dchunk,
            (zero, (zero,) * _TOPK, (zero,) * _TOPK),
            unroll=4)

        qqs = jnp.sum(qq)
        dotv = _lanes([jnp.sum(d) for d in dots], 0.0, iv, jnp.float32)
        ccv = _lanes([jnp.sum(n) for n in nrm], 1.0, iv, jnp.float32)

        s2 = ccv * qqs
        y = lax.bitcast_convert_type(
            jnp.int32(0x5F3759DF) - (lax.bitcast_convert_type(s2, jnp.int32) >> 1),
            jnp.float32)
        for _ in range(4):
            y = y * (jnp.float32(1.5) - jnp.float32(0.5) * s2 * y * y)
        denom = jnp.maximum(s2 * y, jnp.float32(1e-8))
        sim = dotv / denom
        o_v[...] = jnp.where(iv < _TOPK, sim, jnp.float32(0.0))
        pltpu.sync_copy(o_v, out_hbm.at[b])


def _sc_stage(sums, qsums, ctx2d, b_lo):
    B, Lc = sums.shape
    D = qsums.shape[1]
    sums = sums.reshape(B * Lc)
    qsums = qsums.reshape(B * D)
    mesh = plsc.VectorSubcoreMesh(core_axis_name="c", subcore_axis_name="s")
    body = functools.partial(_sc_body, B, Lc, D, b_lo, ctx2d.shape[0])
    cp = pltpu.CompilerParams()
    if "needs_layout_passes" in pltpu.CompilerParams.__dataclass_fields__:
        cp = dataclasses.replace(cp, needs_layout_passes=False)
    kfn = pl.kernel(
        body,
        out_type=jax.ShapeDtypeStruct((B, _L), jnp.float32),
        mesh=mesh,
        compiler_params=cp,
        scratch_types=[
            pltpu.VMEM((Lc,), jnp.float32),
            pltpu.VMEM((D,), jnp.float32),
            pltpu.VMEM((_L,), jnp.int32),
            pltpu.VMEM((_L, D), jnp.float32),
            pltpu.VMEM((_L,), jnp.float32),
            pltpu.SemaphoreType.DMA,
        ],
    )
    return kfn(sums, qsums, ctx2d)


def kernel(question_emb, context_emb, cross_attn_weights):
    B, Lq, D = question_emb.shape
    Lc = context_emb.shape[1]
    attn3 = cross_attn_weights.reshape(B, -1, Lc)
    ctx2d = context_emb.reshape(B * Lc, D)
    s_h, q_h = _tc_reduce(attn3, question_emb, n_chunks=4, b_lo=0, nb=B)
    sims = _sc_stage(s_h.reshape(B, Lc), q_h.reshape(B, D),
                     ctx2d, 0)
    per_batch = 1.0 - jnp.sum(sims, axis=1) / _TOPK
    return jnp.mean(per_batch)
